# baseline (speedup 1.0000x reference)
import jax
import jax.numpy as jnp
from jax.experimental import pallas as pl

_B = 64
_N_IN = 512
_N_OUT = 1024


def _insert_kernel(rho_ref, out_ref):
    i = pl.program_id(1)
    j = pl.program_id(2)

    @pl.when(jnp.logical_and(i == 1, j == 1))
    def _copy():
        out_ref[0] = rho_ref[0]

    @pl.when(jnp.logical_or(i == 0, j == 0))
    def _zero():
        out_ref[0] = jnp.zeros((_N_IN, _N_IN), jnp.float32)


def kernel(rho):
    return pl.pallas_call(
        _insert_kernel,
        grid=(_B, 2, 2),
        in_specs=[pl.BlockSpec((1, _N_IN, _N_IN), lambda b, i, j: (b, 0, 0))],
        out_specs=pl.BlockSpec((1, _N_IN, _N_IN), lambda b, i, j: (b, i, j)),
        out_shape=jax.ShapeDtypeStruct((_B, _N_OUT, _N_OUT), jnp.float32),
    )(rho)


# speedup vs baseline: 2.1089x; 2.1089x over previous
import jax
import jax.numpy as jnp
from jax.experimental import pallas as pl

_B = 64
_N_IN = 512
_N_OUT = 1024


_BB = 4


def _insert_kernel(rho_ref, out_ref):
    out_ref[:, :_N_IN, :] = jnp.zeros((_BB, _N_IN, _N_OUT), jnp.float32)
    out_ref[:, _N_IN:, :_N_IN] = jnp.zeros((_BB, _N_IN, _N_IN), jnp.float32)
    out_ref[:, _N_IN:, _N_IN:] = rho_ref[...]


def kernel(rho):
    return pl.pallas_call(
        _insert_kernel,
        grid=(_B // _BB,),
        in_specs=[pl.BlockSpec((_BB, _N_IN, _N_IN), lambda b: (b, 0, 0))],
        out_specs=pl.BlockSpec((_BB, _N_OUT, _N_OUT), lambda b: (b, 0, 0)),
        out_shape=jax.ShapeDtypeStruct((_B, _N_OUT, _N_OUT), jnp.float32),
    )(rho)
